# baseline (speedup 1.0000x reference)
import functools

import jax
import jax.numpy as jnp
from jax import lax
from jax.experimental import pallas as pl
from jax.experimental.pallas import tpu as pltpu
from jax.experimental.pallas import tpu_sc as plsc

N = 8192
BLOCK_C = 512
CSC = 512
CTC = N - CSC
NW = 32
QPW = N // NW
QV = 4


def _tc_kernel(bn2_ref, at_ref, out_ref):
    step = pl.program_id(0)

    bn2 = bn2_ref[...]
    at = at_ref[...]
    b2 = 0.25 * jnp.sum(bn2 * bn2, axis=1, keepdims=True)
    ut = jnp.dot(bn2, at, preferred_element_type=jnp.float32)
    u = ut + b2
    m = u
    size = BLOCK_C
    while size > 8:
        half = size // 2
        m = jnp.minimum(m[0:half, :], m[half:size, :])
        size = half

    @pl.when(step == 0)
    def _init():
        out_ref[...] = m

    @pl.when(step != 0)
    def _acc():
        out_ref[...] = jnp.minimum(out_ref[...], m)


def _comb_kernel(acc_ref, sc_ref, at_ref, out_ref):
    at = at_ref[...]
    a2 = jnp.sum(at * at, axis=0, keepdims=True)
    m = jnp.min(acc_ref[...], axis=0, keepdims=True)
    m = jnp.minimum(m, sc_ref[...])
    dist = jnp.maximum(a2 + m, 0.0)
    mask = dist <= 2.0
    s = jnp.sum(jnp.where(mask, dist, 0.0))
    c = jnp.sum(mask.astype(jnp.float32))
    out_ref[...] = jnp.reshape(s / jnp.maximum(c, 1.0), (1, 1))


def _sc_body(q_hbm, b_hbm, out_hbm, q_v, b_v, b2_v, o_v):
    cid = lax.axis_index("c")
    sid = lax.axis_index("s")
    wid = sid * 2 + cid
    base = wid * QPW
    pltpu.sync_copy(q_hbm.at[:, pl.ds(base, QPW)], q_v)
    pltpu.sync_copy(b_hbm, b_v)

    def b2_step(t, carry):
        sl = pl.ds(t * 16, 16)
        bx = b_v[3, sl]
        by = b_v[4, sl]
        bz = b_v[5, sl]
        b2_v[sl] = bx * bx + by * by + bz * bz
        return carry

    lax.fori_loop(0, CSC // 16, b2_step, 0)

    for p in range(QPW // (16 * QV)):
        qs = []
        for g in range(QV):
            sl = pl.ds((p * QV + g) * 16, 16)
            qs.append((q_v[0, sl], q_v[1, sl], q_v[2, sl]))
        init = tuple(jnp.full((16,), jnp.inf, jnp.float32) for _ in range(QV))

        def j_step(t, ms):
            sl = pl.ds(t * 16, 16)
            bx16 = b_v[0, sl]
            by16 = b_v[1, sl]
            bz16 = b_v[2, sl]
            b216 = b2_v[sl]
            ms = list(ms)
            for l in range(16):
                bxj = bx16[l]
                byj = by16[l]
                bzj = bz16[l]
                b2j = b216[l]
                for g, (qx, qy, qz) in enumerate(qs):
                    u = b2j + qx * bxj
                    u = u + qy * byj
                    u = u + qz * bzj
                    ms[g] = jnp.minimum(ms[g], u)
            return tuple(ms)

        ms = lax.fori_loop(0, CSC // 16, j_step, init)
        for g in range(QV):
            o_v[pl.ds((p * QV + g) * 16, 16)] = ms[g]

    pltpu.sync_copy(o_v, out_hbm.at[pl.ds(base, QPW)])


_sc_min = functools.partial(
    pl.kernel,
    mesh=plsc.VectorSubcoreMesh(core_axis_name="c", subcore_axis_name="s"),
    out_type=jax.ShapeDtypeStruct((N,), jnp.float32),
    scratch_types=[
        pltpu.VMEM((3, QPW), jnp.float32),
        pltpu.VMEM((6, CSC), jnp.float32),
        pltpu.VMEM((CSC,), jnp.float32),
        pltpu.VMEM((QPW,), jnp.float32),
    ],
)(_sc_body)


@jax.jit
def _nn(at, bn2_tc, qpack, bpack):
    acc8 = pl.pallas_call(
        _tc_kernel,
        grid=(CTC // BLOCK_C,),
        in_specs=[
            pl.BlockSpec((BLOCK_C, 8), lambda i: (i, 0)),
            pl.BlockSpec((8, N), lambda i: (0, 0)),
        ],
        out_specs=pl.BlockSpec((8, N), lambda i: (0, 0)),
        out_shape=jax.ShapeDtypeStruct((8, N), jnp.float32),
    )(bn2_tc, at)
    scmin = _sc_min(qpack, bpack)
    out = pl.pallas_call(
        _comb_kernel,
        in_specs=[
            pl.BlockSpec((8, N), lambda: (0, 0)),
            pl.BlockSpec((1, N), lambda: (0, 0)),
            pl.BlockSpec((8, N), lambda: (0, 0)),
        ],
        out_specs=pl.BlockSpec((1, 1), lambda: (0, 0)),
        out_shape=jax.ShapeDtypeStruct((1, 1), jnp.float32),
    )(acc8, scmin.reshape(1, N), at)
    return out[0, 0]


def kernel(input0, input1):
    at = jnp.zeros((8, N), jnp.float32).at[:3, :].set(input0.T)
    bn2_tc = jnp.zeros((CTC, 8), jnp.float32).at[:, :3].set(-2.0 * input1[:CTC])
    qpack = input0.T.astype(jnp.bfloat16).astype(jnp.float32)
    b_sc = input1[CTC:]
    bn2_sc = (-2.0 * b_sc.T.astype(jnp.bfloat16).astype(jnp.float32))
    bpack = jnp.concatenate([bn2_sc, b_sc.T], axis=0)
    return _nn(at, bn2_tc, qpack, bpack)


# speedup vs baseline: 1.9223x; 1.9223x over previous
import functools

import jax
import jax.numpy as jnp
from jax.experimental import pallas as pl
from jax.experimental.pallas import tpu as pltpu

N = 8192
BLOCK_C = 1024


def _nn_kernel(bn2_ref, at_ref, out_ref, acc_ref):
    step = pl.program_id(0)

    bn2 = bn2_ref[...]
    at = at_ref[...]
    b2 = 0.25 * jnp.sum(bn2 * bn2, axis=1, keepdims=True)
    ut = jnp.dot(bn2, at, preferred_element_type=jnp.float32)
    u = ut + b2
    m = u
    size = BLOCK_C
    while size > 8:
        half = size // 2
        m = jnp.minimum(m[0:half, :], m[half:size, :])
        size = half

    @pl.when(step == 0)
    def _init():
        acc_ref[...] = m

    @pl.when(step != 0)
    def _acc():
        acc_ref[...] = jnp.minimum(acc_ref[...], m)

    @pl.when(step == pl.num_programs(0) - 1)
    def _fin():
        a2 = jnp.sum(at * at, axis=0, keepdims=True)
        mfull = jnp.min(acc_ref[...], axis=0, keepdims=True)
        dist = jnp.maximum(a2 + mfull, 0.0)
        mask = dist <= 2.0
        s = jnp.sum(jnp.where(mask, dist, 0.0))
        c = jnp.sum(mask.astype(jnp.float32))
        out_ref[...] = jnp.reshape(s / jnp.maximum(c, 1.0), (1, 1))


@jax.jit
def _nn(bn2, at):
    out = pl.pallas_call(
        _nn_kernel,
        grid=(N // BLOCK_C,),
        in_specs=[
            pl.BlockSpec((BLOCK_C, 8), lambda i: (i, 0)),
            pl.BlockSpec((8, N), lambda i: (0, 0)),
        ],
        out_specs=pl.BlockSpec((1, 1), lambda i: (0, 0)),
        out_shape=jax.ShapeDtypeStruct((1, 1), jnp.float32),
        scratch_shapes=[
            pltpu.VMEM((8, N), jnp.float32),
        ],
    )(bn2, at)
    return out[0, 0]


def kernel(input0, input1):
    bn2 = jnp.zeros((N, 8), jnp.float32).at[:, :3].set(-2.0 * input1)
    at = jnp.zeros((8, N), jnp.float32).at[:3, :].set(input0.T)
    return _nn(bn2, at)


# speedup vs baseline: 1.9492x; 1.0140x over previous
import functools

import jax
import jax.numpy as jnp
from jax.experimental import pallas as pl
from jax.experimental.pallas import tpu as pltpu

N = 8192
BLOCK_C = 2048


def _nn_kernel(bn2_ref, at_ref, out_ref, acc_ref):
    step = pl.program_id(0)

    bn2 = bn2_ref[...]
    at = at_ref[...]
    b2 = 0.25 * jnp.sum(bn2 * bn2, axis=1, keepdims=True)
    ut = jnp.dot(bn2, at, preferred_element_type=jnp.float32)
    u = ut + b2
    m = u
    size = BLOCK_C
    while size > 8:
        half = size // 2
        m = jnp.minimum(m[0:half, :], m[half:size, :])
        size = half

    @pl.when(step == 0)
    def _init():
        acc_ref[...] = m

    @pl.when(step != 0)
    def _acc():
        acc_ref[...] = jnp.minimum(acc_ref[...], m)

    @pl.when(step == pl.num_programs(0) - 1)
    def _fin():
        a2 = jnp.sum(at * at, axis=0, keepdims=True)
        mfull = jnp.min(acc_ref[...], axis=0, keepdims=True)
        dist = jnp.maximum(a2 + mfull, 0.0)
        mask = dist <= 2.0
        s = jnp.sum(jnp.where(mask, dist, 0.0))
        c = jnp.sum(mask.astype(jnp.float32))
        out_ref[...] = jnp.reshape(s / jnp.maximum(c, 1.0), (1, 1))


@jax.jit
def _nn(bn2, at):
    out = pl.pallas_call(
        _nn_kernel,
        grid=(N // BLOCK_C,),
        in_specs=[
            pl.BlockSpec((BLOCK_C, 8), lambda i: (i, 0)),
            pl.BlockSpec((8, N), lambda i: (0, 0)),
        ],
        out_specs=pl.BlockSpec((1, 1), lambda i: (0, 0)),
        out_shape=jax.ShapeDtypeStruct((1, 1), jnp.float32),
        scratch_shapes=[
            pltpu.VMEM((8, N), jnp.float32),
        ],
    )(bn2, at)
    return out[0, 0]


def kernel(input0, input1):
    bn2 = jnp.zeros((N, 8), jnp.float32).at[:, :3].set(-2.0 * input1)
    at = jnp.zeros((8, N), jnp.float32).at[:3, :].set(input0.T)
    return _nn(bn2, at)
